# Initial kernel scaffold; baseline (speedup 1.0000x reference)
import jax
import jax.numpy as jnp
from jax.experimental import pallas as pl


def kernel(X_inputs, nn_idx):
    raise NotImplementedError("write your pallas kernel here")


# speedup vs baseline: 2.0279x; 2.0279x over previous
import functools

import jax
import jax.numpy as jnp
from jax import lax
from jax.experimental import pallas as pl
from jax.experimental.pallas import tpu as pltpu
from jax.experimental.pallas import tpu_sc as plsc

_L = 16


@functools.partial(jax.jit, static_argnums=(2, 3, 4, 5))
def _edge_sc(x_flat, idx_flat, BN, D, K, G):
    NC, NS = 2, 16
    NW = NC * NS
    NPW = BN // NW
    GR = G * K
    n_groups = NPW // G

    mesh = plsc.VectorSubcoreMesh(core_axis_name="c", subcore_axis_name="s")

    @functools.partial(
        pl.kernel,
        mesh=mesh,
        out_type=jax.ShapeDtypeStruct((BN * K, 2 * D), jnp.float32),
        scratch_types=[
            pltpu.VMEM((GR,), jnp.int32),
            pltpu.VMEM((GR, D), jnp.float32),
            pltpu.VMEM((G, D), jnp.float32),
            pltpu.VMEM((GR, 2 * D), jnp.float32),
            pltpu.SemaphoreType.DMA,
        ],
    )
    def k(x_hbm, idx_hbm, out_hbm, idx_v, nbr_v, ctr_v, out_v, sem):
        wid = lax.axis_index("s") * NC + lax.axis_index("c")
        n0 = wid * NPW

        def group_body(g, carry):
            nbase = n0 + g * G
            rbase = nbase * K
            pltpu.sync_copy(idx_hbm.at[pl.ds(rbase, GR)], idx_v)
            gather = pltpu.async_copy(x_hbm.at[idx_v], nbr_v, sem)
            pltpu.sync_copy(x_hbm.at[pl.ds(nbase, G)], ctr_v)
            gather.wait()

            def row_body(r, carry2):
                i = r // K
                for j in range(D // _L):
                    c = ctr_v[i, pl.ds(j * _L, _L)]
                    nv = nbr_v[r, pl.ds(j * _L, _L)]
                    out_v[r, pl.ds(j * _L, _L)] = c
                    out_v[r, pl.ds(D + j * _L, _L)] = nv - c
                return carry2

            lax.fori_loop(0, GR, row_body, 0)
            pltpu.sync_copy(out_v, out_hbm.at[pl.ds(rbase, GR)])
            return carry

        lax.fori_loop(0, n_groups, group_body, 0)

    return k(x_flat, idx_flat)


def kernel(X_inputs, nn_idx):
    B, N, D = X_inputs.shape
    K = nn_idx.shape[-1]
    x_flat = X_inputs.reshape(B * N, D)
    offs = (jnp.arange(B, dtype=jnp.int32) * N).reshape(B, 1, 1)
    idx_flat = (nn_idx.astype(jnp.int32) + offs).reshape(B * N * K)
    out = _edge_sc(x_flat, idx_flat, B * N, D, K, 8)
    return out.reshape(B, N, K, 2 * D)
